# baseline (speedup 1.0000x reference)
import jax
import jax.numpy as jnp
from jax.experimental import pallas as pl


_BS = 256


def _add_body(x_ref, pos_ref, o_ref):
    p = pos_ref[...]
    b = x_ref.shape[1] // p.shape[1]
    o_ref[...] = x_ref[...] + jnp.concatenate([p] * b, axis=1)


def kernel(x, pos_table):
    s, batch, d = x.shape
    x2 = x.reshape(s, batch * d)
    out = pl.pallas_call(
        _add_body,
        grid=(s // _BS,),
        in_specs=[
            pl.BlockSpec((_BS, batch * d), lambda i: (i, 0)),
            pl.BlockSpec((_BS, d), lambda i: (i, 0)),
        ],
        out_specs=pl.BlockSpec((_BS, batch * d), lambda i: (i, 0)),
        out_shape=jax.ShapeDtypeStruct((s, batch * d), x.dtype),
    )(x2, pos_table)
    return out.reshape(s, batch, d)


# speedup vs baseline: 3.9349x; 3.9349x over previous
import jax
import jax.numpy as jnp
from jax.experimental import pallas as pl


_BS = 256


def _add_body(x_ref, pos_ref, o_ref):
    o_ref[...] = x_ref[...] + pos_ref[...][:, None, :]


def kernel(x, pos_table):
    s, batch, d = x.shape
    return pl.pallas_call(
        _add_body,
        grid=(s // _BS,),
        in_specs=[
            pl.BlockSpec((_BS, batch, d), lambda i: (i, 0, 0)),
            pl.BlockSpec((_BS, d), lambda i: (i, 0)),
        ],
        out_specs=pl.BlockSpec((_BS, batch, d), lambda i: (i, 0, 0)),
        out_shape=jax.ShapeDtypeStruct((s, batch, d), x.dtype),
    )(x, pos_table)


# speedup vs baseline: 3.9937x; 1.0149x over previous
import jax
import jax.numpy as jnp
from jax.experimental import pallas as pl


_BS = 512


def _add_body(x_ref, pos_ref, o_ref):
    o_ref[...] = x_ref[...] + pos_ref[...][:, None, :]


def kernel(x, pos_table):
    s, batch, d = x.shape
    return pl.pallas_call(
        _add_body,
        grid=(s // _BS,),
        in_specs=[
            pl.BlockSpec((_BS, batch, d), lambda i: (i, 0, 0)),
            pl.BlockSpec((_BS, d), lambda i: (i, 0)),
        ],
        out_specs=pl.BlockSpec((_BS, batch, d), lambda i: (i, 0, 0)),
        out_shape=jax.ShapeDtypeStruct((s, batch, d), x.dtype),
    )(x, pos_table)


# speedup vs baseline: 4.0065x; 1.0032x over previous
import jax
import jax.numpy as jnp
from jax.experimental import pallas as pl


_BS = 1024
_BD = 512


def _add_body(x_ref, pos_ref, o_ref):
    o_ref[...] = x_ref[...] + pos_ref[...][:, None, :]


def kernel(x, pos_table):
    s, batch, d = x.shape
    return pl.pallas_call(
        _add_body,
        grid=(s // _BS, d // _BD),
        in_specs=[
            pl.BlockSpec((_BS, batch, _BD), lambda i, j: (i, 0, j)),
            pl.BlockSpec((_BS, _BD), lambda i, j: (i, j)),
        ],
        out_specs=pl.BlockSpec((_BS, batch, _BD), lambda i, j: (i, 0, j)),
        out_shape=jax.ShapeDtypeStruct((s, batch, d), x.dtype),
    )(x, pos_table)
